# Initial kernel scaffold; baseline (speedup 1.0000x reference)
import jax
import jax.numpy as jnp
from jax.experimental import pallas as pl


def kernel(center, pos, neg, in_emb, out_emb):
    raise NotImplementedError("write your pallas kernel here")


# speedup vs baseline: 5.2373x; 5.2373x over previous
import functools

import jax
import jax.numpy as jnp
from jax import lax
from jax.experimental import pallas as pl
from jax.experimental.pallas import tpu as pltpu
from jax.experimental.pallas import tpu_sc as plsc

VOCAB = 1000000
DIM = 64
B = 16384
NNEG = 20

NC = 2
NS = 16
NW = NC * NS
BPW = B // NW
C = 64
NCHUNK = BPW // C
NEGC = C * NNEG
NEG_GROUPS = NEGC // 128
LANES = 16
NV = DIM // LANES
SLOTS = 32


def _merge(a, b, sh):
    perm = lax.iota(jnp.int32, LANES) ^ sh
    mask = (lax.iota(jnp.int32, LANES) & sh) == 0
    return (jnp.where(mask, a, jnp.take(b, perm))
            + jnp.where(mask, jnp.take(a, perm), b))


def _merge_half(a, sh):
    perm = lax.iota(jnp.int32, LANES) ^ sh
    mask = (lax.iota(jnp.int32, LANES) & sh) == 0
    return jnp.where(mask, a + jnp.take(a, perm), jnp.zeros((LANES,), jnp.float32))


def _reduce16(accs):
    sh = 1
    while len(accs) > 1 or sh <= 8:
        nxt = []
        for i in range(0, len(accs), 2):
            if i + 1 < len(accs):
                nxt.append(_merge(accs[i], accs[i + 1], sh))
            else:
                nxt.append(_merge_half(accs[i], sh))
        accs = nxt
        sh *= 2
        if sh > 8:
            break
    out = accs[0]
    while sh <= 8:
        out = _merge_half(out, sh)
        sh *= 2
    return out


def _sc_scores(center, pos, negflat, in_emb, out_emb):
    mesh = plsc.VectorSubcoreMesh(core_axis_name="c", subcore_axis_name="s")

    @functools.partial(
        pl.kernel,
        out_type=jax.ShapeDtypeStruct((B * SLOTS,), jnp.float32),
        mesh=mesh,
        scratch_types=[
            pltpu.VMEM((C,), jnp.int32),
            pltpu.VMEM((C,), jnp.int32),
            pltpu.VMEM((NEGC,), jnp.int32),
            pltpu.VMEM((C, DIM), jnp.float32),
            pltpu.VMEM((C, DIM), jnp.float32),
            pltpu.VMEM((NEGC, DIM), jnp.float32),
            pltpu.VMEM((C * SLOTS,), jnp.float32),
            pltpu.SemaphoreType.DMA,
        ],
        compiler_params=pltpu.CompilerParams(use_tc_tiling_on_sc=False),
    )
    def k(center_hbm, pos_hbm, neg_hbm, inemb_hbm, outemb_hbm, scores_hbm,
          cidx, pidx, nidx, vrow, urow, nrow, srow, sem):
        wid = lax.axis_index("s") * NC + lax.axis_index("c")

        def chunk_body(ci, carry):
            base = wid * BPW + ci * C
            pltpu.sync_copy(center_hbm.at[pl.ds(base, C)], cidx)
            pltpu.sync_copy(pos_hbm.at[pl.ds(base, C)], pidx)
            pltpu.sync_copy(neg_hbm.at[pl.ds(base * NNEG, NEGC)], nidx)

            cps = [
                pltpu.async_copy(inemb_hbm.at[cidx], vrow, sem),
                pltpu.async_copy(outemb_hbm.at[pidx], urow, sem),
            ]
            for j in range(NEG_GROUPS):
                cps.append(pltpu.async_copy(
                    outemb_hbm.at[nidx.at[pl.ds(j * 128, 128)]],
                    nrow.at[pl.ds(j * 128, 128)], sem))
            for cp in cps:
                cp.wait()

            def dot_acc(vv, row_ref, r):
                acc = vv[0] * row_ref[r, pl.ds(0, LANES)]
                for i in range(1, NV):
                    acc = acc + vv[i] * row_ref[r, pl.ds(i * LANES, LANES)]
                return acc

            def b_body(b, carry2):
                vv = [vrow[b, pl.ds(i * LANES, LANES)] for i in range(NV)]
                accs_a = [dot_acc(vv, urow, b)]
                for kk in range(15):
                    accs_a.append(dot_acc(vv, nrow, b * NNEG + kk))
                srow[pl.ds(b * SLOTS, LANES)] = _reduce16(accs_a)
                accs_b = [dot_acc(vv, nrow, b * NNEG + kk) for kk in range(15, NNEG)]
                srow[pl.ds(b * SLOTS + LANES, LANES)] = _reduce16(accs_b)
                return carry2

            lax.fori_loop(0, C, b_body, 0)
            pltpu.sync_copy(srow, scores_hbm.at[pl.ds(base * SLOTS, C * SLOTS)])
            return carry

        lax.fori_loop(0, NCHUNK, chunk_body, 0)

    return k(center, pos, negflat, in_emb, out_emb)


def _tc_loss(scores2d):
    def body(s_ref, o_ref):
        s = s_ref[...]
        slot = lax.rem(lax.broadcasted_iota(jnp.int32, s.shape, 1), SLOTS)
        x = jnp.where(slot == 0, s, -s)
        w = jnp.where(slot == 0, 1.0 / B,
                      jnp.where(slot <= NNEG, 1.0 / (B * NNEG), 0.0))
        total = jnp.sum(jax.nn.log_sigmoid(x) * w)
        o_ref[...] = jnp.full((1, 1), -total, dtype=jnp.float32)

    return pl.pallas_call(
        body,
        out_shape=jax.ShapeDtypeStruct((1, 1), jnp.float32),
    )(scores2d)


def kernel(center, pos, neg, in_emb, out_emb):
    center = center.astype(jnp.int32)
    pos = pos.astype(jnp.int32)
    negflat = neg.astype(jnp.int32).reshape(B * NNEG)
    scores = _sc_scores(center, pos, negflat, in_emb, out_emb)
    out = _tc_loss(scores.reshape(B * SLOTS // 128, 128))
    return out.reshape(())


# speedup vs baseline: 8.9016x; 1.6996x over previous
import functools

import jax
import jax.numpy as jnp
from jax import lax
from jax.experimental import pallas as pl
from jax.experimental.pallas import tpu as pltpu
from jax.experimental.pallas import tpu_sc as plsc

VOCAB = 1000000
DIM = 64
B = 16384
NNEG = 20

NC = 2
NS = 16
NW = NC * NS
BPW = B // NW
C = 32
NCHUNK = BPW // C
NEGC = C * NNEG
NEG_GROUPS = NEGC // 128
LANES = 16
NV = DIM // LANES
SLOTS = 32
PD = 128


def _merge(a, b, sh):
    perm = lax.iota(jnp.int32, LANES) ^ sh
    mask = (lax.iota(jnp.int32, LANES) & sh) == 0
    return (jnp.where(mask, a, jnp.take(b, perm))
            + jnp.where(mask, jnp.take(a, perm), b))


def _merge_half(a, sh):
    perm = lax.iota(jnp.int32, LANES) ^ sh
    mask = (lax.iota(jnp.int32, LANES) & sh) == 0
    return jnp.where(mask, a + jnp.take(a, perm), jnp.zeros((LANES,), jnp.float32))


def _reduce16(accs):
    sh = 1
    while len(accs) > 1 or sh <= 8:
        nxt = []
        for i in range(0, len(accs), 2):
            if i + 1 < len(accs):
                nxt.append(_merge(accs[i], accs[i + 1], sh))
            else:
                nxt.append(_merge_half(accs[i], sh))
        accs = nxt
        sh *= 2
        if sh > 8:
            break
    out = accs[0]
    while sh <= 8:
        out = _merge_half(out, sh)
        sh *= 2
    return out


def _sc_scores(center, pos, negflat, in_emb, out_emb):
    mesh = plsc.VectorSubcoreMesh(core_axis_name="c", subcore_axis_name="s")

    @functools.partial(
        pl.kernel,
        out_type=jax.ShapeDtypeStruct((B * SLOTS,), jnp.float32),
        mesh=mesh,
        scratch_types=[
            pltpu.VMEM((C,), jnp.int32),
            pltpu.VMEM((C,), jnp.int32),
            pltpu.VMEM((NEGC,), jnp.int32),
            pltpu.VMEM((C, PD), jnp.float32),
            pltpu.VMEM((C, PD), jnp.float32),
            pltpu.VMEM((NEGC, PD), jnp.float32),
            pltpu.VMEM((C * SLOTS,), jnp.float32),
            pltpu.SemaphoreType.DMA,
        ],
        compiler_params=pltpu.CompilerParams(use_tc_tiling_on_sc=False),
    )
    def k(center_hbm, pos_hbm, neg_hbm, inemb_hbm, outemb_hbm, scores_hbm,
          cidx, pidx, nidx, vrow, urow, nrow, srow, sem):
        wid = lax.axis_index("s") * NC + lax.axis_index("c")

        def chunk_body(ci, carry):
            base = wid * BPW + ci * C
            pltpu.sync_copy(center_hbm.at[pl.ds(base, C)], cidx)
            pltpu.sync_copy(pos_hbm.at[pl.ds(base, C)], pidx)
            pltpu.sync_copy(neg_hbm.at[pl.ds(base * NNEG, NEGC)], nidx)

            cps = [
                pltpu.async_copy(inemb_hbm.at[cidx], vrow, sem),
                pltpu.async_copy(outemb_hbm.at[pidx], urow, sem),
            ]
            for j in range(NEG_GROUPS):
                cps.append(pltpu.async_copy(
                    outemb_hbm.at[nidx.at[pl.ds(j * 128, 128)]],
                    nrow.at[pl.ds(j * 128, 128)], sem))
            for cp in cps:
                cp.wait()

            def dot_acc(vv, row_ref, r):
                acc = vv[0] * row_ref[r, pl.ds(0, LANES)]
                for i in range(1, NV):
                    acc = acc + vv[i] * row_ref[r, pl.ds(i * LANES, LANES)]
                return acc

            def b_body(b, carry2):
                vv = [vrow[b, pl.ds(i * LANES, LANES)] for i in range(NV)]
                accs_a = [dot_acc(vv, urow, b)]
                for kk in range(15):
                    accs_a.append(dot_acc(vv, nrow, b * NNEG + kk))
                srow[pl.ds(b * SLOTS, LANES)] = _reduce16(accs_a)
                accs_b = [dot_acc(vv, nrow, b * NNEG + kk) for kk in range(15, NNEG)]
                srow[pl.ds(b * SLOTS + LANES, LANES)] = _reduce16(accs_b)
                return carry2

            lax.fori_loop(0, C, b_body, 0)
            pltpu.sync_copy(srow, scores_hbm.at[pl.ds(base * SLOTS, C * SLOTS)])
            return carry

        lax.fori_loop(0, NCHUNK, chunk_body, 0)

    return k(center, pos, negflat, in_emb, out_emb)


TBLK = 8192


def _tc_padT(table_t):
    def body(x_ref, o_ref):
        o_ref[:, 0:DIM] = jnp.transpose(x_ref[...])

    return pl.pallas_call(
        body,
        grid=(pl.cdiv(VOCAB, TBLK),),
        in_specs=[pl.BlockSpec((DIM, TBLK), lambda g: (0, g))],
        out_specs=pl.BlockSpec((TBLK, PD), lambda g: (g, 0)),
        out_shape=jax.ShapeDtypeStruct((VOCAB, PD), jnp.float32),
    )(table_t)


def _tc_loss(scores2d):
    def body(s_ref, o_ref):
        s = s_ref[...]
        slot = lax.rem(lax.broadcasted_iota(jnp.int32, s.shape, 1), SLOTS)
        x = jnp.where(slot == 0, s, -s)
        w = jnp.where(slot == 0, 1.0 / B,
                      jnp.where(slot <= NNEG, 1.0 / (B * NNEG), 0.0))
        total = jnp.sum(jax.nn.log_sigmoid(x) * w)
        o_ref[...] = jnp.full((1, 1), -total, dtype=jnp.float32)

    return pl.pallas_call(
        body,
        out_shape=jax.ShapeDtypeStruct((1, 1), jnp.float32),
    )(scores2d)


def kernel(center, pos, neg, in_emb, out_emb):
    center = center.astype(jnp.int32)
    pos = pos.astype(jnp.int32)
    negflat = neg.astype(jnp.int32).reshape(B * NNEG)
    in_p = _tc_padT(in_emb.T)
    out_p = _tc_padT(out_emb.T)
    scores = _sc_scores(center, pos, negflat, in_p, out_p)
    out = _tc_loss(scores.reshape(B * SLOTS // 128, 128))
    return out.reshape(())


# speedup vs baseline: 9.7644x; 1.0969x over previous
import functools

import jax
import jax.numpy as jnp
from jax import lax
from jax.experimental import pallas as pl
from jax.experimental.pallas import tpu as pltpu
from jax.experimental.pallas import tpu_sc as plsc

VOCAB = 1000000
DIM = 64
B = 16384
NNEG = 20

NC = 2
NS = 16
NW = NC * NS
BPW = B // NW
C = 16
NCHUNK = BPW // C
NEGC = C * NNEG
NEG_G = 4
NEG_GSZ = NEGC // NEG_G
LANES = 16
SLOTS = 32
PD = 128


def _merge(a, b, sh):
    perm = lax.iota(jnp.int32, LANES) ^ sh
    mask = (lax.iota(jnp.int32, LANES) & sh) == 0
    return (jnp.where(mask, a, jnp.take(b, perm))
            + jnp.where(mask, jnp.take(a, perm), b))


def _merge_half(a, sh):
    perm = lax.iota(jnp.int32, LANES) ^ sh
    mask = (lax.iota(jnp.int32, LANES) & sh) == 0
    return jnp.where(mask, a + jnp.take(a, perm), jnp.zeros((LANES,), jnp.float32))


def _reduce16(accs):
    sh = 1
    while len(accs) > 1 or sh <= 8:
        nxt = []
        for i in range(0, len(accs), 2):
            if i + 1 < len(accs):
                nxt.append(_merge(accs[i], accs[i + 1], sh))
            else:
                nxt.append(_merge_half(accs[i], sh))
        accs = nxt
        sh *= 2
        if sh > 8:
            break
    out = accs[0]
    while sh <= 8:
        out = _merge_half(out, sh)
        sh *= 2
    return out


def _sc_scores(center, pos, negflat, in_p, out_p):
    mesh = plsc.VectorSubcoreMesh(core_axis_name="c", subcore_axis_name="s")

    @functools.partial(
        pl.kernel,
        out_type=jax.ShapeDtypeStruct((B * SLOTS,), jnp.float32),
        mesh=mesh,
        scratch_types=[
            pltpu.VMEM((BPW,), jnp.int32),
            pltpu.VMEM((BPW,), jnp.int32),
            pltpu.VMEM((BPW * NNEG,), jnp.int32),
            pltpu.VMEM((C, PD), jnp.float32),
            pltpu.VMEM((C, PD), jnp.float32),
            pltpu.VMEM((NEGC, PD), jnp.float32),
            pltpu.VMEM((C * SLOTS,), jnp.float32),
            pltpu.VMEM((C, PD), jnp.float32),
            pltpu.VMEM((C, PD), jnp.float32),
            pltpu.VMEM((NEGC, PD), jnp.float32),
            pltpu.VMEM((C * SLOTS,), jnp.float32),
            pltpu.SemaphoreType.DMA,
            pltpu.SemaphoreType.DMA,
            pltpu.SemaphoreType.DMA,
            pltpu.SemaphoreType.DMA,
        ],
    )
    def k(center_hbm, pos_hbm, neg_hbm, inp_hbm, outp_hbm, scores_hbm,
          cidxa, pidxa, nidxa,
          vrow0, urow0, nrow0, srow0, vrow1, urow1, nrow1, srow1,
          sem0, sem1, sems0, sems1):
        wid = lax.axis_index("s") * NC + lax.axis_index("c")
        wbase = wid * BPW
        pltpu.sync_copy(center_hbm.at[pl.ds(wbase, BPW)], cidxa)
        pltpu.sync_copy(pos_hbm.at[pl.ds(wbase, BPW)], pidxa)
        pltpu.sync_copy(neg_hbm.at[pl.ds(wbase * NNEG, BPW * NNEG)], nidxa)

        bufs = [(vrow0, urow0, nrow0, srow0, sem0, sems0),
                (vrow1, urow1, nrow1, srow1, sem1, sems1)]

        def start(c, bi):
            vrow, urow, nrow, _, sem, _ = bufs[bi]
            pltpu.async_copy(inp_hbm.at[cidxa.at[pl.ds(c * C, C)]], vrow, sem)
            pltpu.async_copy(outp_hbm.at[pidxa.at[pl.ds(c * C, C)]], urow, sem)
            for g in range(NEG_G):
                pltpu.async_copy(
                    outp_hbm.at[nidxa.at[pl.ds(c * NEGC + g * NEG_GSZ,
                                               NEG_GSZ)]],
                    nrow.at[pl.ds(g * NEG_GSZ, NEG_GSZ)], sem)

        def wait_rows(bi):
            vrow, urow, nrow, _, sem, _ = bufs[bi]
            pltpu.make_async_copy(inp_hbm.at[pl.ds(0, C)], vrow, sem).wait()
            pltpu.make_async_copy(inp_hbm.at[pl.ds(0, C)], urow, sem).wait()
            pltpu.make_async_copy(inp_hbm.at[pl.ds(0, NEGC)], nrow, sem).wait()

        def compute(c, bi):
            vrow, urow, nrow, srow, _, sems = bufs[bi]

            @pl.when(c >= 2)
            def _():
                pltpu.make_async_copy(
                    srow, scores_hbm.at[pl.ds(0, C * SLOTS)], sems).wait()

            def dot_acc(vv, row_ref, r):
                acc = vv[0] * row_ref[r, pl.ds(0, LANES)]
                for i in range(1, 4):
                    acc = acc + vv[i] * row_ref[r, pl.ds(i * LANES, LANES)]
                return acc

            def b_body(b, carry2):
                vv = [vrow[b, pl.ds(i * LANES, LANES)] for i in range(4)]
                accs_a = [dot_acc(vv, urow, b)]
                for kk in range(15):
                    accs_a.append(dot_acc(vv, nrow, b * NNEG + kk))
                srow[pl.ds(b * SLOTS, LANES)] = _reduce16(accs_a)
                accs_b = [dot_acc(vv, nrow, b * NNEG + kk)
                          for kk in range(15, NNEG)]
                srow[pl.ds(b * SLOTS + LANES, LANES)] = _reduce16(accs_b)
                return carry2

            lax.fori_loop(0, C, b_body, 0)
            base = wbase + c * C
            pltpu.async_copy(
                srow, scores_hbm.at[pl.ds(base * SLOTS, C * SLOTS)], sems)

        start(0, 0)

        def pair_body(p, carry):
            c0 = 2 * p
            start(c0 + 1, 1)
            wait_rows(0)
            compute(c0, 0)

            @pl.when(c0 + 2 < NCHUNK)
            def _():
                start(c0 + 2, 0)

            wait_rows(1)
            compute(c0 + 1, 1)
            return carry

        lax.fori_loop(0, NCHUNK // 2, pair_body, 0)
        pltpu.make_async_copy(
            srow0, scores_hbm.at[pl.ds(0, C * SLOTS)], sems0).wait()
        pltpu.make_async_copy(
            srow1, scores_hbm.at[pl.ds(0, C * SLOTS)], sems1).wait()

    return k(center, pos, negflat, in_p, out_p)


TBLK = 8192


def _tc_padT(table_t):
    def body(x_ref, o_ref):
        o_ref[:, 0:DIM] = jnp.transpose(x_ref[...])

    return pl.pallas_call(
        body,
        grid=(pl.cdiv(VOCAB, TBLK),),
        in_specs=[pl.BlockSpec((DIM, TBLK), lambda g: (0, g))],
        out_specs=pl.BlockSpec((TBLK, PD), lambda g: (g, 0)),
        out_shape=jax.ShapeDtypeStruct((VOCAB, PD), jnp.float32),
    )(table_t)


def _tc_loss(scores2d):
    def body(s_ref, o_ref):
        s = s_ref[...]
        slot = lax.rem(lax.broadcasted_iota(jnp.int32, s.shape, 1), SLOTS)
        x = jnp.where(slot == 0, s, -s)
        w = jnp.where(slot == 0, 1.0 / B,
                      jnp.where(slot <= NNEG, 1.0 / (B * NNEG), 0.0))
        total = jnp.sum(jax.nn.log_sigmoid(x) * w)
        o_ref[...] = jnp.full((1, 1), -total, dtype=jnp.float32)

    return pl.pallas_call(
        body,
        out_shape=jax.ShapeDtypeStruct((1, 1), jnp.float32),
    )(scores2d)


def kernel(center, pos, neg, in_emb, out_emb):
    center = center.astype(jnp.int32)
    pos = pos.astype(jnp.int32)
    negflat = neg.astype(jnp.int32).reshape(B * NNEG)
    in_p = _tc_padT(in_emb.T)
    out_p = _tc_padT(out_emb.T)
    scores = _sc_scores(center, pos, negflat, in_p, out_p)
    out = _tc_loss(scores.reshape(B * SLOTS // 128, 128))
    return out.reshape(())
